# baseline (speedup 1.0000x reference)
import functools
import numpy as np
import jax
import jax.numpy as jnp
from jax import lax
from jax.experimental import pallas as pl
from jax.experimental.pallas import tpu as pltpu
from jax.experimental.pallas import tpu_sc as plsc

_C, _D, _H, _W = 4, 128, 224, 224
_NS = 64
_DBLK = 32
_NK = _D // _DBLK

_AX = np.linspace(0, _D - 1, _NS).astype(np.int32)
_SG = np.linspace(0, _W - 1, _NS).astype(np.int32)
_CO = np.linspace(0, _H - 1, _NS).astype(np.int32)
assert all(int(_SG[s]) == (s * (_W - 1)) // (_NS - 1) for s in range(_NS))
assert all(int(_CO[s]) == (s * (_H - 1)) // (_NS - 1) for s in range(_NS))
assert all(int(_AX[s]) == (2 * s if s < 63 else 127) for s in range(_NS))

_NW = 32
_SPW = (_C * _NS) // _NW


def _ax_body(vol, ax, buf):
    wid = lax.axis_index("s") * 2 + lax.axis_index("c")

    def do_i(i, _):
        sl = wid * _SPW + i
        c = sl // _NS
        s = sl % _NS
        d = jnp.where(s == _NS - 1, _D - 1, 2 * s)
        pltpu.sync_copy(vol.at[c, d], buf)
        pltpu.sync_copy(buf, ax.at[c, s])
        return 0

    lax.fori_loop(0, _SPW, do_i, 0)


def _sc_axial(volume):
    mesh = plsc.VectorSubcoreMesh(core_axis_name="c", subcore_axis_name="s")
    k = functools.partial(
        pl.kernel,
        mesh=mesh,
        out_type=jax.ShapeDtypeStruct((_C, _NS, _H, _W), jnp.float32),
        scratch_types=[pltpu.VMEM((_H, _W), jnp.float32)],
        compiler_params=pltpu.CompilerParams(needs_layout_passes=False),
    )(_ax_body)
    return k(volume)


def _tc_body(vol_ref, sag_ref, cor_ref, tp_ref):
    for p in range(_DBLK):
        tp_ref[p] = vol_ref[0, p].T
    for s in range(_NS):
        sag_ref[0, s, :, :] = tp_ref[:, int(_SG[s]), :]
    for s in range(_NS):
        cor_ref[0, s, :, :] = vol_ref[0, :, int(_CO[s]), :]


def _tc_sag_cor(volume):
    return pl.pallas_call(
        _tc_body,
        grid=(_C, _NK),
        in_specs=[
            pl.BlockSpec((1, _DBLK, _H, _W), lambda c, k: (c, k, 0, 0)),
        ],
        out_specs=[
            pl.BlockSpec((1, _NS, _DBLK, _H), lambda c, k: (c, 0, k, 0)),
            pl.BlockSpec((1, _NS, _DBLK, _W), lambda c, k: (c, 0, k, 0)),
        ],
        out_shape=[
            jax.ShapeDtypeStruct((_C, _NS, _D, _H), jnp.float32),
            jax.ShapeDtypeStruct((_C, _NS, _D, _W), jnp.float32),
        ],
        scratch_shapes=[pltpu.VMEM((_DBLK, _W, _H), jnp.float32)],
        compiler_params=pltpu.CompilerParams(
            dimension_semantics=("parallel", "parallel")),
    )(volume)


@jax.jit
def kernel(volume):
    axial = _sc_axial(volume)
    sagittal, coronal = _tc_sag_cor(volume)
    return (axial, sagittal, coronal)
